# Initial kernel scaffold; baseline (speedup 1.0000x reference)
import jax
import jax.numpy as jnp
from jax.experimental import pallas as pl


def kernel(feats_node, edge_index, feats_graph, W1, b1, W2, b2, W3, b3, LW1, Lb1, LW2, Lb2, LW3, Lb3):
    raise NotImplementedError("write your pallas kernel here")


# speedup vs baseline: 4.3653x; 4.3653x over previous
import functools

import jax
import jax.numpy as jnp
from jax import lax
from jax.experimental import pallas as pl
from jax.experimental.pallas import tpu as pltpu
from jax.experimental.pallas import tpu_sc as plsc

N = 10000
E = 320000
F = 128
NPAD = 10240
_NC, _NS = 2, 16
_TILES = _NC * _NS
_C = 80

_SELU_ALPHA = 1.6732632423543772
_SELU_SCALE = 1.0507009873554805


def _selu(x):
    xn = jnp.minimum(x, 0.0)
    u = jnp.exp(xn)
    um1 = u - 1.0
    lu = jnp.log(u)
    em1 = jnp.where(lu == 0.0, xn, um1 * xn / jnp.where(lu == 0.0, 1.0, lu))
    return _SELU_SCALE * jnp.where(x > 0, x, _SELU_ALPHA * em1)


def _sc_degrees(src, dst):
    mesh = plsc.VectorSubcoreMesh(core_axis_name="c", subcore_axis_name="s")
    EPT = E // _NS
    NCHUNK = EPT // _C
    ROWS = NPAD // _NS

    @functools.partial(
        pl.kernel,
        out_type=jax.ShapeDtypeStruct((2, NPAD, 16), jnp.float32),
        mesh=mesh,
        scratch_types=[
            pltpu.VMEM((_C,), jnp.int32),
            pltpu.VMEM((_C, 16), jnp.float32),
            pltpu.VMEM((ROWS, 16), jnp.float32),
            pltpu.VMEM_SHARED((NPAD, 16), jnp.float32),
            pltpu.SemaphoreType.DMA,
        ],
        compiler_params=pltpu.CompilerParams(use_tc_tiling_on_sc=False),
    )
    def k(src_hbm, dst_hbm, out_hbm, idx_v, val_v, zbuf_v, acc_sh, sem):
        cid = lax.axis_index("c")
        sid = lax.axis_index("s")
        lanes = lax.iota(jnp.int32, 16)
        onehot = jnp.where(lanes == 0, jnp.float32(1.0), jnp.float32(0.0))
        zero16 = jnp.zeros((16,), jnp.float32)

        def fill_val(i, c):
            val_v[i, :] = onehot
            return c
        lax.fori_loop(0, _C, fill_val, 0)

        def fill_zero(i, c):
            zbuf_v[i, :] = zero16
            return c
        lax.fori_loop(0, ROWS, fill_zero, 0)

        pltpu.sync_copy(zbuf_v, acc_sh.at[pl.ds(sid * ROWS, ROWS)])
        plsc.subcore_barrier()

        base = sid * EPT

        def chunk(i, c):
            off = base + i * _C

            @pl.when(cid == 0)
            def _():
                pltpu.sync_copy(src_hbm.at[pl.ds(off, _C)], idx_v)

            @pl.when(cid == 1)
            def _():
                pltpu.sync_copy(dst_hbm.at[pl.ds(off, _C)], idx_v)

            pltpu.sync_copy(val_v, acc_sh.at[idx_v], add=True)
            return c
        lax.fori_loop(0, NCHUNK, chunk, 0)

        plsc.subcore_barrier()
        pltpu.sync_copy(acc_sh.at[pl.ds(sid * ROWS, ROWS)],
                        out_hbm.at[cid, pl.ds(sid * ROWS, ROWS)])

    return k(src, dst)


def _sc_msgpass(h, src, dst):
    mesh = plsc.VectorSubcoreMesh(core_axis_name="c", subcore_axis_name="s")
    EPT = E // _TILES
    NCHUNK = EPT // _C
    ROWS = NPAD // _NS
    ZR = 64

    @functools.partial(
        pl.kernel,
        out_type=jax.ShapeDtypeStruct((2, NPAD, F), jnp.float32),
        mesh=mesh,
        scratch_types=[
            pltpu.VMEM((_C,), jnp.int32),
            pltpu.VMEM((_C,), jnp.int32),
            pltpu.VMEM((_C, F), jnp.float32),
            pltpu.VMEM((ZR, F), jnp.float32),
            pltpu.VMEM_SHARED((NPAD, F), jnp.float32),
            pltpu.SemaphoreType.DMA,
        ],
        compiler_params=pltpu.CompilerParams(use_tc_tiling_on_sc=False),
    )
    def k(h_hbm, src_hbm, dst_hbm, out_hbm, src_v, dst_v, rows_v, zbuf_v,
          acc_sh, sem):
        cid = lax.axis_index("c")
        sid = lax.axis_index("s")
        zero16 = jnp.zeros((16,), jnp.float32)

        def fill_zero(i, c):
            for j in range(F // 16):
                zbuf_v[i, pl.ds(j * 16, 16)] = zero16
            return c
        lax.fori_loop(0, ZR, fill_zero, 0)

        def zero_acc(t, c):
            pltpu.sync_copy(zbuf_v, acc_sh.at[pl.ds(sid * ROWS + t * ZR, ZR)])
            return c
        lax.fori_loop(0, ROWS // ZR, zero_acc, 0)
        plsc.subcore_barrier()

        base = (cid * _NS + sid) * EPT

        def chunk(i, c):
            off = base + i * _C
            pltpu.sync_copy(src_hbm.at[pl.ds(off, _C)], src_v)
            pltpu.sync_copy(dst_hbm.at[pl.ds(off, _C)], dst_v)
            pltpu.async_copy(h_hbm.at[src_v], rows_v, sem).wait()
            pltpu.sync_copy(rows_v, acc_sh.at[dst_v], add=True)
            return c
        lax.fori_loop(0, NCHUNK, chunk, 0)

        plsc.subcore_barrier()
        pltpu.sync_copy(acc_sh.at[pl.ds(sid * ROWS, ROWS)],
                        out_hbm.at[cid, pl.ds(sid * ROWS, ROWS)])

    return k(h, src, dst)


_RB = 1000
_G = N // _RB


def _tc_rsqrt(deg):
    def body(d_ref, r_ref):
        r_ref[...] = lax.rsqrt(jnp.maximum(d_ref[...], 1.0))
    return pl.pallas_call(
        body, out_shape=jax.ShapeDtypeStruct((2, NPAD, 16), jnp.float32))(deg)


def _tc_scale(x, r_out):
    def body(x_ref, r_ref, o_ref):
        o_ref[...] = x_ref[...] * r_ref[...]
    return pl.pallas_call(
        body,
        grid=(_G,),
        in_specs=[pl.BlockSpec((_RB, F), lambda i: (i, 0)),
                  pl.BlockSpec((_RB, 1), lambda i: (i, 0))],
        out_specs=pl.BlockSpec((_RB, F), lambda i: (i, 0)),
        out_shape=jax.ShapeDtypeStruct((N, F), jnp.float32))(x, r_out)


def _tc_conv_out(p, r_in, r_out, W, b):
    def body(p_ref, ri_ref, ro_ref, w_ref, b_ref, o_ref):
        agg = (p_ref[0] + p_ref[1]) * ri_ref[...]
        x = _selu(jnp.dot(agg, w_ref[...],
                          preferred_element_type=jnp.float32) + b_ref[...])
        o_ref[...] = x * ro_ref[...]
    return pl.pallas_call(
        body,
        grid=(_G,),
        in_specs=[pl.BlockSpec((2, _RB, F), lambda i: (0, i, 0)),
                  pl.BlockSpec((_RB, 1), lambda i: (i, 0)),
                  pl.BlockSpec((_RB, 1), lambda i: (i, 0)),
                  pl.BlockSpec((F, F), lambda i: (0, 0)),
                  pl.BlockSpec((1, F), lambda i: (0, 0))],
        out_specs=pl.BlockSpec((_RB, F), lambda i: (i, 0)),
        out_shape=jax.ShapeDtypeStruct((N, F), jnp.float32))(p, r_in, r_out, W, b)


def _tc_readout(p, r_in, W, b):
    def body(p_ref, ri_ref, w_ref, b_ref, o_ref):
        agg = (p_ref[0] + p_ref[1]) * ri_ref[...]
        y = jnp.dot(agg, w_ref[...],
                    preferred_element_type=jnp.float32) + b_ref[...]
        part = jnp.sum(y, axis=0, keepdims=True)

        @pl.when(pl.program_id(0) == 0)
        def _():
            o_ref[...] = jnp.zeros_like(o_ref)
        o_ref[...] += part
    return pl.pallas_call(
        body,
        grid=(_G,),
        in_specs=[pl.BlockSpec((2, _RB, F), lambda i: (0, i, 0)),
                  pl.BlockSpec((_RB, 1), lambda i: (i, 0)),
                  pl.BlockSpec((F, F), lambda i: (0, 0)),
                  pl.BlockSpec((1, F), lambda i: (0, 0))],
        out_specs=pl.BlockSpec((1, F), lambda i: (0, 0)),
        out_shape=jax.ShapeDtypeStruct((1, F), jnp.float32))(p, r_in, W, b)


def _tc_head(ysum, fg, LW1, Lb1, LW2, Lb2, LW3, Lb3):
    def body(y_ref, fg_ref, l1_ref, lb1_ref, l2_ref, lb2_ref, l3_ref,
             lb3_ref, o_ref):
        u = jnp.dot(y_ref[...], l1_ref[0:F, :], preferred_element_type=jnp.float32)
        for j in range(3):
            u += fg_ref[0, j] * l1_ref[pl.ds(F + j, 1), :]
        u = _selu(u + lb1_ref[...])
        v = _selu(jnp.dot(u, l2_ref[...],
                          preferred_element_type=jnp.float32) + lb2_ref[...])
        o_ref[...] = jnp.dot(v, l3_ref[...],
                             preferred_element_type=jnp.float32) + lb3_ref[...]
    return pl.pallas_call(
        body, out_shape=jax.ShapeDtypeStruct((1, 1), jnp.float32))(
            ysum, fg, LW1, Lb1, LW2, Lb2, LW3, Lb3)


def kernel(feats_node, edge_index, feats_graph, W1, b1, W2, b2, W3, b3,
           LW1, Lb1, LW2, Lb2, LW3, Lb3):
    ei = edge_index.astype(jnp.int32)
    src = ei[0]
    dst = ei[1]

    deg = _sc_degrees(src, dst)
    R = _tc_rsqrt(deg)
    r_out = R[0, :N, 0:1]
    r_in = R[1, :N, 0:1]

    xs1 = _tc_scale(feats_node, r_out)
    p1 = _sc_msgpass(xs1, src, dst)
    xs2 = _tc_conv_out(p1, r_in, r_out, W1, b1.reshape(1, F))
    p2 = _sc_msgpass(xs2, src, dst)
    xs3 = _tc_conv_out(p2, r_in, r_out, W2, b2.reshape(1, F))
    p3 = _sc_msgpass(xs3, src, dst)
    ysum = _tc_readout(p3, r_in, W3, b3.reshape(1, F))

    out = _tc_head(ysum, feats_graph, LW1, Lb1.reshape(1, -1), LW2,
                   Lb2.reshape(1, -1), LW3, Lb3.reshape(1, 1))
    return out.reshape(-1)
